# baseline (speedup 1.0000x reference)
import functools

import jax
import jax.numpy as jnp
from jax import lax
from jax.experimental import pallas as pl
from jax.experimental.pallas import tpu as pltpu
from jax.experimental.pallas import tpu_sc as plsc

VOCAB = 1000000
DIM = 64
SEQ = 200
BATCH = 4096

NC = 2
NS = 16
NW = NC * NS
BPW = BATCH // NW
NCH = SEQ
NBUF = 4
LANES = 16
DBLK = DIM // 8
TILEW = 8 * BPW
VREGS = DIM * BPW // LANES


def _sc_body(xt, tab, pos, out, idx_all, pos_v, bufs, tbufs, gsems, osems):
    wid = lax.axis_index("s") * NC + lax.axis_index("c")
    b0 = wid * BPW

    pltpu.sync_copy(xt.at[:, pl.ds(b0, BPW)], idx_all)
    pltpu.sync_copy(pos, pos_v)

    iota = lax.iota(jnp.int32, LANES)

    def start_gather(c, b):
        pltpu.async_copy(tab.at[idx_all.at[c]], bufs[b], gsems[b])

    def wait_gather(b):
        pltpu.make_async_copy(tab.at[pl.ds(0, BPW)], bufs[b], gsems[b]).wait()

    def start_out(c, b):
        for dblk in range(DBLK):
            pltpu.async_copy(
                tbufs[b].at[dblk],
                out.at[pl.ds(((c * DBLK + dblk) * NW + wid) * TILEW, TILEW)],
                osems[b])

    def wait_out(b):
        for dblk in range(DBLK):
            pltpu.make_async_copy(tbufs[b].at[dblk],
                                  out.at[pl.ds(0, TILEW)], osems[b]).wait()

    for b in range(NBUF - 1):
        start_gather(b, b)

    def group_body(g, carry):
        for b in range(NBUF):
            c = g * NBUF + b
            f = c + NBUF - 1
            bf = (b + NBUF - 1) % NBUF

            @pl.when(jnp.logical_and(f >= NBUF, f < NCH))
            def _():
                wait_out(bf)
                start_gather(f, bf)

            @pl.when(jnp.logical_and(f < NBUF, f < NCH))
            def _():
                start_gather(f, bf)

            wait_gather(b)

            buf = bufs[b]
            tbuf = tbufs[b]

            def vbody(d, carry2):
                cols = jnp.full((LANES,), d, jnp.int32)
                crow = jnp.full((LANES,), c, jnp.int32)
                pv = plsc.load_gather(pos_v, [crow, cols])
                for gg in range(BPW // LANES):
                    rows = iota + gg * LANES
                    v = plsc.load_gather(buf, [rows, cols])
                    tbuf[d // 8,
                         pl.ds((d % 8) * BPW + gg * LANES, LANES)] = v + pv
                return carry2

            lax.fori_loop(0, DIM, vbody, 0, unroll=2)

            start_out(c, b)
        return carry

    lax.fori_loop(0, NCH // NBUF, group_body, 0)

    for b in range(NBUF):
        wait_out(b)


@jax.jit
def _sc_call(xt, token_table, pos_table):
    mesh = plsc.VectorSubcoreMesh(core_axis_name="c", subcore_axis_name="s")

    def body(xt_r, tab_r, pos_r, out_r, idx_all, pos_v,
             b0, b1, b2, b3, t0, t1, t2, t3,
             g0, g1, g2, g3, o0, o1, o2, o3):
        _sc_body(xt_r, tab_r, pos_r, out_r, idx_all, pos_v,
                 [b0, b1, b2, b3], [t0, t1, t2, t3],
                 [g0, g1, g2, g3], [o0, o1, o2, o3])

    return pl.kernel(
        body,
        out_type=jax.ShapeDtypeStruct((SEQ * BATCH * DIM,), jnp.float32),
        mesh=mesh,
        compiler_params=pltpu.CompilerParams(use_tc_tiling_on_sc=False,
                                             needs_layout_passes=False),
        scratch_types=(
            [pltpu.VMEM((SEQ, BPW), jnp.int32),
             pltpu.VMEM((SEQ, DIM), jnp.float32)]
            + [pltpu.VMEM((BPW, DIM), jnp.float32) for _ in range(NBUF)]
            + [pltpu.VMEM((DBLK, TILEW), jnp.float32) for _ in range(NBUF)]
            + [pltpu.SemaphoreType.DMA for _ in range(2 * NBUF)]
        ),
    )(xt, token_table, pos_table)


def kernel(x, token_table, pos_table):
    xt = x.astype(jnp.int32).T
    out1d = _sc_call(xt, token_table, pos_table)
    return (out1d.reshape(SEQ, DBLK, NW, 8, BPW)
            .transpose(2, 4, 0, 1, 3)
            .reshape(BATCH, SEQ, DIM))


# speedup vs baseline: 1.6292x; 1.6292x over previous
import functools

import jax
import jax.numpy as jnp
from jax import lax
from jax.experimental import pallas as pl
from jax.experimental.pallas import tpu as pltpu
from jax.experimental.pallas import tpu_sc as plsc

VOCAB = 1000000
DIM = 64
SEQ = 200
BATCH = 4096

NC = 2
NS = 16
NW = NC * NS
ROWS = BATCH * SEQ
BPW = BATCH // NW
NCH = SEQ
NBUF = 4
LANES = 16


def _sc_body(xt, tab, pos, out, idx_all, pos_v, bufs, sidxs, gsems, osems):
    wid = lax.axis_index("s") * NC + lax.axis_index("c")
    b0 = wid * BPW

    pltpu.sync_copy(xt.at[:, pl.ds(b0, BPW)], idx_all)
    pltpu.sync_copy(pos, pos_v)

    iota = lax.iota(jnp.int32, LANES)

    def start_gather(c, b):
        pltpu.async_copy(tab.at[idx_all.at[c]], bufs[b], gsems[b])

    def wait_gather(b):
        pltpu.make_async_copy(tab.at[pl.ds(0, BPW)], bufs[b], gsems[b]).wait()

    def start_out(c, b):
        sidx = sidxs[b]
        for g in range(BPW // LANES):
            sidx[pl.ds(g * LANES, LANES)] = (
                (iota + (b0 + g * LANES)) * SEQ + c)
        pltpu.async_copy(bufs[b], out.at[sidx], osems[b])

    def wait_out(b):
        pltpu.make_async_copy(bufs[b], out.at[pl.ds(0, BPW)], osems[b]).wait()

    for b in range(NBUF - 1):
        start_gather(b, b)

    def group_body(g, carry):
        for b in range(NBUF):
            c = g * NBUF + b
            f = c + NBUF - 1
            bf = (b + NBUF - 1) % NBUF

            @pl.when(jnp.logical_and(f >= NBUF, f < NCH))
            def _():
                wait_out(bf)
                start_gather(f, bf)

            @pl.when(jnp.logical_and(f < NBUF, f < NCH))
            def _():
                start_gather(f, bf)

            wait_gather(b)

            buf = bufs[b]
            prow = [pos_v[c, pl.ds(d * LANES, LANES)]
                    for d in range(DIM // LANES)]

            def jbody(j, carry2):
                for d in range(DIM // LANES):
                    s = pl.ds(d * LANES, LANES)
                    buf[j, s] = buf[j, s] + prow[d]
                return carry2

            lax.fori_loop(0, BPW, jbody, 0, unroll=4)

            start_out(c, b)
        return carry

    lax.fori_loop(0, NCH // NBUF, group_body, 0)

    for b in range(NBUF):
        wait_out(b)


@jax.jit
def _sc_call(xt, token_table, pos_table):
    mesh = plsc.VectorSubcoreMesh(core_axis_name="c", subcore_axis_name="s")

    def body(xt_r, tab_r, pos_r, out_r, idx_all, pos_v,
             b0, b1, b2, b3, s0, s1, s2, s3,
             g0, g1, g2, g3, o0, o1, o2, o3):
        _sc_body(xt_r, tab_r, pos_r, out_r, idx_all, pos_v,
                 [b0, b1, b2, b3], [s0, s1, s2, s3],
                 [g0, g1, g2, g3], [o0, o1, o2, o3])

    return pl.kernel(
        body,
        out_type=jax.ShapeDtypeStruct((ROWS, DIM), jnp.float32),
        mesh=mesh,
        compiler_params=pltpu.CompilerParams(use_tc_tiling_on_sc=False,
                                             needs_layout_passes=False),
        scratch_types=(
            [pltpu.VMEM((SEQ, BPW), jnp.int32),
             pltpu.VMEM((SEQ, DIM), jnp.float32)]
            + [pltpu.VMEM((BPW, DIM), jnp.float32) for _ in range(NBUF)]
            + [pltpu.VMEM((BPW,), jnp.int32) for _ in range(NBUF)]
            + [pltpu.SemaphoreType.DMA for _ in range(2 * NBUF)]
        ),
    )(xt, token_table, pos_table)


def kernel(x, token_table, pos_table):
    xt = x.astype(jnp.int32).T
    out = _sc_call(xt, token_table, pos_table)
    return out.reshape(BATCH, SEQ, DIM)


# speedup vs baseline: 1.6752x; 1.0282x over previous
import functools

import jax
import jax.numpy as jnp
from jax import lax
from jax.experimental import pallas as pl
from jax.experimental.pallas import tpu as pltpu
from jax.experimental.pallas import tpu_sc as plsc

VOCAB = 1000000
DIM = 64
SEQ = 200
BATCH = 4096

NC = 2
NS = 16
NW = NC * NS
BPW = BATCH // NW
NCH = SEQ
NBUF = 4
LANES = 16


def _sc_body(xt, tab, pos, out, idx_all, pos_v, bufs, gsems, osems):
    wid = lax.axis_index("s") * NC + lax.axis_index("c")
    b0 = wid * BPW

    pltpu.sync_copy(xt.at[:, pl.ds(b0, BPW)], idx_all)
    pltpu.sync_copy(pos, pos_v)

    def start_gather(c, b):
        pltpu.async_copy(tab.at[idx_all.at[c]], bufs[b], gsems[b])

    def wait_gather(b):
        pltpu.make_async_copy(tab.at[pl.ds(0, BPW)], bufs[b], gsems[b]).wait()

    def start_out(c, b):
        pltpu.async_copy(bufs[b], out.at[c, pl.ds(b0, BPW)], osems[b])

    def wait_out(b):
        pltpu.make_async_copy(bufs[b], out.at[0, pl.ds(b0, BPW)],
                              osems[b]).wait()

    for b in range(NBUF - 1):
        start_gather(b, b)

    def group_body(g, carry):
        for b in range(NBUF):
            c = g * NBUF + b
            f = c + NBUF - 1
            bf = (b + NBUF - 1) % NBUF

            @pl.when(jnp.logical_and(f >= NBUF, f < NCH))
            def _():
                wait_out(bf)
                start_gather(f, bf)

            @pl.when(jnp.logical_and(f < NBUF, f < NCH))
            def _():
                start_gather(f, bf)

            wait_gather(b)

            buf = bufs[b]
            prow = [pos_v[c, pl.ds(d * LANES, LANES)]
                    for d in range(DIM // LANES)]

            def jbody(j, carry2):
                for d in range(DIM // LANES):
                    s = pl.ds(d * LANES, LANES)
                    buf[j, s] = buf[j, s] + prow[d]
                return carry2

            lax.fori_loop(0, BPW, jbody, 0, unroll=4)

            start_out(c, b)
        return carry

    lax.fori_loop(0, NCH // NBUF, group_body, 0)

    for b in range(NBUF):
        wait_out(b)


@jax.jit
def _sc_call(xt, token_table, pos_table):
    mesh = plsc.VectorSubcoreMesh(core_axis_name="c", subcore_axis_name="s")

    def body(xt_r, tab_r, pos_r, out_r, idx_all, pos_v,
             b0, b1, b2, b3, g0, g1, g2, g3, o0, o1, o2, o3):
        _sc_body(xt_r, tab_r, pos_r, out_r, idx_all, pos_v,
                 [b0, b1, b2, b3], [g0, g1, g2, g3], [o0, o1, o2, o3])

    return pl.kernel(
        body,
        out_type=jax.ShapeDtypeStruct((SEQ, BATCH, DIM), jnp.float32),
        mesh=mesh,
        compiler_params=pltpu.CompilerParams(use_tc_tiling_on_sc=False),
        scratch_types=(
            [pltpu.VMEM((SEQ, BPW), jnp.int32),
             pltpu.VMEM((SEQ, DIM), jnp.float32)]
            + [pltpu.VMEM((BPW, DIM), jnp.float32) for _ in range(NBUF)]
            + [pltpu.SemaphoreType.DMA for _ in range(2 * NBUF)]
        ),
    )(xt, token_table, pos_table)


def kernel(x, token_table, pos_table):
    xt = x.astype(jnp.int32).T
    out_t = _sc_call(xt, token_table, pos_table)
    return out_t.transpose(1, 0, 2)
